# baseline (speedup 1.0000x reference)
import jax
import jax.numpy as jnp
import numpy as np
from jax.experimental import pallas as pl

N = 10000
HID = 64
HEADS = 4
NUM_LAYERS = 3
NUM_CLASSES = 8
TEMP = 2.0


def _layernorm(x, g, b):
    mu = jnp.mean(x, axis=-1, keepdims=True)
    var = jnp.mean((x - mu) ** 2, axis=-1, keepdims=True)
    return (x - mu) / jnp.sqrt(var + 1e-5) * g + b


def _conv(h, src, dst, edge_emb, c):
    q = (h @ c['Wq'] + c['bq']).reshape(N, HEADS, HID)
    k = (h @ c['Wk'] + c['bk']).reshape(N, HEADS, HID)
    v = (h @ c['Wv'] + c['bv']).reshape(N, HEADS, HID)
    e = (edge_emb @ c['We']).reshape(-1, HEADS, HID)
    kj = k[src] + e
    vj = v[src] + e
    alpha = jnp.sum(q[dst] * kj, axis=-1) / jnp.sqrt(float(HID))
    amax = jax.ops.segment_max(alpha, dst, num_segments=N)
    amax = jnp.where(jnp.isfinite(amax), amax, 0.0)
    ex = jnp.exp(alpha - amax[dst])
    denom = jax.ops.segment_sum(ex, dst, num_segments=N)
    attn = ex / (denom[dst] + 1e-16)
    msg = vj * attn[..., None]
    out = jax.ops.segment_sum(msg, dst, num_segments=N).reshape(N, HEADS * HID)
    out = out + h @ c['Ws'] + c['bs']
    return out


def _logits_kernel(h_ref, w_ref, b_ref, o_ref):
    o_ref[...] = (h_ref[...] @ w_ref[...] + b_ref[...]) / TEMP


def kernel(x, edge_index, edge_attr, params):
    src = edge_index[0]
    dst = edge_index[1]
    h = x @ params['node_W'] + params['node_b']
    ee = edge_attr @ params['edge_W'] + params['edge_b']
    for i in range(NUM_LAYERS):
        c = params['convs'][i]
        hn = _conv(h, src, dst, ee, c)
        hn = hn @ params['proj_W'] + params['proj_b']
        hn = _layernorm(hn, params['ln_g'][i], params['ln_b'][i])
        h = h + hn
        if i < NUM_LAYERS - 1:
            h = jax.nn.relu(h)
    target = h[0:1]
    out = pl.pallas_call(
        _logits_kernel,
        out_shape=jax.ShapeDtypeStruct((8, NUM_CLASSES), jnp.float32),
    )(jnp.broadcast_to(target, (8, HID)), params['cls_W'],
      jnp.broadcast_to(params['cls_b'], (8, NUM_CLASSES)))
    return out[0:1]


# speedup vs baseline: 3.1856x; 3.1856x over previous
import functools

import jax
import jax.numpy as jnp
from jax import lax
from jax.experimental import pallas as pl
from jax.experimental.pallas import tpu as pltpu
from jax.experimental.pallas import tpu_sc as plsc

N = 10000
E = 160000
F_IN = 33
HID = 64
HEADS = 4
NUM_LAYERS = 3
NUM_CLASSES = 8
EDGE_DIM = 4
TEMP = 2.0

NPAD = 10240
EPAD = 163840
NW = 32
PERW = EPAD // NW
CHUNK = 128
NCH = PERW // CHUNK
ACCW = 128
HALF = 5120
AROWS = 6144
ROWS_PER_SUB = AROWS // 16
DUMMY = N


@functools.cache
def _mesh():
    return plsc.VectorSubcoreMesh(core_axis_name="c", subcore_axis_name="s")


def _dense_body(x_ref, w_ref, b_ref, o_ref):
    o_ref[...] = x_ref[...] @ w_ref[...] + b_ref[0:1, :]


def _dense(xp, w, b, rb):
    rows, cin = xp.shape
    cout = w.shape[1]
    return pl.pallas_call(
        _dense_body,
        grid=(rows // rb,),
        in_specs=[
            pl.BlockSpec((rb, cin), lambda i: (i, 0)),
            pl.BlockSpec((cin, cout), lambda i: (0, 0)),
            pl.BlockSpec((8, cout), lambda i: (0, 0)),
        ],
        out_specs=pl.BlockSpec((rb, cout), lambda i: (i, 0)),
        out_shape=jax.ShapeDtypeStruct((rows, cout), jnp.float32),
    )(xp, w, jnp.broadcast_to(b, (8, cout)))


def _qkv_body(h_ref, wq, wk, wv, ws, we, ewp, bq, bk, bv, bs, *outs):
    h = h_ref[...]
    q = h @ wq[...] + bq[0:1, :]
    k = h @ wk[...] + bk[0:1, :]
    v = h @ wv[...] + bv[0:1, :]
    outs[8][...] = h @ ws[...] + bs[0:1, :]
    we_a = we[...]
    ewp_a = ewp[...]
    rb = h.shape[0]
    for hh in range(HEADS):
        sl = slice(hh * HID, (hh + 1) * HID)
        qh = q[:, sl]
        qeh = lax.dot_general(qh, we_a[:, sl], (((1,), (1,)), ((), ())))
        qab = lax.dot_general(qeh, ewp_a, (((1,), (1,)), ((), ())))
        outs[hh][:, 0:HID] = qh
        outs[hh][:, HID:HID + 16] = qab
        outs[hh][:, HID + 16:2 * HID] = jnp.zeros((rb, 48), jnp.float32)
        outs[4 + hh][:, 0:HID] = k[:, sl]
        outs[4 + hh][:, HID:2 * HID] = v[:, sl]


def _qkv(h, c, ewp):
    rb = 512
    row = lambda i: (i, 0)
    full = lambda i: (0, 0)
    outs = pl.pallas_call(
        _qkv_body,
        grid=(NPAD // rb,),
        in_specs=[pl.BlockSpec((rb, HID), row)]
        + [pl.BlockSpec((HID, HEADS * HID), full)] * 5
        + [pl.BlockSpec((16, HID), full)]
        + [pl.BlockSpec((8, HEADS * HID), full)] * 4,
        out_specs=[pl.BlockSpec((rb, 2 * HID), row)] * 8
        + [pl.BlockSpec((rb, HEADS * HID), row)],
        out_shape=[jax.ShapeDtypeStruct((NPAD, 2 * HID), jnp.float32)] * 8
        + [jax.ShapeDtypeStruct((NPAD, HEADS * HID), jnp.float32)],
    )(
        h, c['Wq'], c['Wk'], c['Wv'], c['Ws'], c['We'], ewp,
        jnp.broadcast_to(c['bq'], (8, HEADS * HID)),
        jnp.broadcast_to(c['bk'], (8, HEADS * HID)),
        jnp.broadcast_to(c['bv'], (8, HEADS * HID)),
        jnp.broadcast_to(c['bs'], (8, HEADS * HID)),
    )
    return outs[0:4], outs[4:8], outs[8]


def _amax_body(p_ref, a_ref, m_ref):
    a = jnp.sum(p_ref[...][0], axis=-1)
    a_ref[...] = a[None, None, :]
    m_ref[...] = jnp.full((1, 1, 1, 128), jnp.max(a), jnp.float32)


def _alpha_max(pacc):
    eb = 2048
    nb = EPAD // eb
    return pl.pallas_call(
        _amax_body,
        grid=(HEADS, nb),
        in_specs=[pl.BlockSpec((1, eb, 16), lambda h, i: (h, i, 0))],
        out_specs=[
            pl.BlockSpec((1, 1, eb), lambda h, i: (h, 0, i)),
            pl.BlockSpec((1, 1, 1, 128), lambda h, i: (h, i, 0, 0)),
        ],
        out_shape=[
            jax.ShapeDtypeStruct((HEADS, 1, EPAD), jnp.float32),
            jax.ShapeDtypeStruct((HEADS, nb, 1, 128), jnp.float32),
        ],
    )(pacc)


def _exp_body(a_ref, m_ref, e_ref):
    gmax = jnp.max(m_ref[...])
    e_ref[...] = jnp.exp(a_ref[...] - gmax)


def _exp(alpha, bmax):
    eb = 2048
    nb = EPAD // eb
    return pl.pallas_call(
        _exp_body,
        grid=(HEADS, nb),
        in_specs=[
            pl.BlockSpec((1, 1, eb), lambda h, i: (h, 0, i)),
            pl.BlockSpec((1, nb, 1, 128), lambda h, i: (h, 0, 0, 0)),
        ],
        out_specs=pl.BlockSpec((1, 1, eb), lambda h, i: (h, 0, i)),
        out_shape=jax.ShapeDtypeStruct((HEADS, 1, EPAD), jnp.float32),
    )(alpha, bmax)


def _m2_body(acc_ref, skip_ref, we, ewp, pw, pb, g, bt, hin_ref, o_ref, *, relu):
    a = acc_ref[...]
    we_a = we[...]
    ewp_a = ewp[...]
    msgs = []
    for hh in range(HEADS):
        asum = a[hh, 0] + a[hh, 1]
        num = asum[:, 0:HID]
        sad = asum[:, HID:HID + 16]
        den = asum[:, HID + EDGE_DIM]
        unfold = ewp_a @ we_a[:, hh * HID:(hh + 1) * HID]
        m = (num + sad @ unfold) / (den[:, None] + 1e-16)
        msgs.append(m)
    out = jnp.concatenate(msgs, axis=1) + skip_ref[...]
    hn = out @ pw[...] + pb[0:1, :]
    mu = jnp.mean(hn, axis=1, keepdims=True)
    var = jnp.mean((hn - mu) ** 2, axis=1, keepdims=True)
    hn = (hn - mu) / jnp.sqrt(var + 1e-5) * g[0:1, :] + bt[0:1, :]
    hv = hin_ref[...] + hn
    if relu:
        hv = jnp.maximum(hv, 0.0)
    o_ref[...] = hv


def _m2(acc, skip, we, ewp, pw, pb, g, bt, hin, relu):
    rb = 512
    row = lambda i: (i, 0)
    full = lambda i: (0, 0)
    return pl.pallas_call(
        functools.partial(_m2_body, relu=relu),
        grid=(NPAD // rb,),
        in_specs=[
            pl.BlockSpec((HEADS, 2, rb, ACCW), lambda i: (0, 0, i, 0)),
            pl.BlockSpec((rb, HEADS * HID), row),
            pl.BlockSpec((HID, HEADS * HID), full),
            pl.BlockSpec((16, HID), full),
            pl.BlockSpec((HEADS * HID, HID), full),
            pl.BlockSpec((8, HID), full),
            pl.BlockSpec((8, HID), full),
            pl.BlockSpec((8, HID), full),
            pl.BlockSpec((rb, HID), row),
        ],
        out_specs=pl.BlockSpec((rb, HID), row),
        out_shape=jax.ShapeDtypeStruct((NPAD, HID), jnp.float32),
    )(acc, skip, we, ewp, pw,
      jnp.broadcast_to(pb, (8, HID)), jnp.broadcast_to(g, (8, HID)),
      jnp.broadcast_to(bt, (8, HID)), hin)


def _logits_body(h_ref, w_ref, b_ref, o_ref):
    o_ref[...] = (h_ref[...] @ w_ref[...] + b_ref[...]) / TEMP


def _logits(target, cw, cb):
    out = pl.pallas_call(
        _logits_body,
        out_shape=jax.ShapeDtypeStruct((8, NUM_CLASSES), jnp.float32),
    )(jnp.broadcast_to(target, (8, HID)), cw,
      jnp.broadcast_to(cb, (8, NUM_CLASSES)))
    return out[0:1]


@functools.cache
def _k1():
    return functools.partial(
        pl.kernel,
        out_type=jax.ShapeDtypeStruct((HEADS, EPAD, 16), jnp.float32),
        mesh=_mesh(),
        scratch_types=[
            pltpu.VMEM((CHUNK,), jnp.int32),
            pltpu.VMEM((CHUNK,), jnp.int32),
            pltpu.VMEM((CHUNK, 16), jnp.float32),
            pltpu.VMEM((CHUNK, 2 * HID), jnp.float32),
            pltpu.VMEM((CHUNK, 2 * HID), jnp.float32),
            pltpu.VMEM((CHUNK, 16), jnp.float32),
            pltpu.SemaphoreType.DMA,
        ],
    )(_k1_body)


def _k1_body(td0, td1, td2, td3, tk0, tk1, tk2, tk3, eah, srch, dsth,
             pacc_out, sidx, didx, eav, rd, rs, pb, sem):
    cid = lax.axis_index("c")
    sid = lax.axis_index("s")
    wid = sid * 2 + cid
    base0 = wid * PERW
    tds = [td0, td1, td2, td3]
    tks = [tk0, tk1, tk2, tk3]

    def chunk_body(t, _):
        base = base0 + t * CHUNK
        pltpu.sync_copy(srch.at[pl.ds(base, CHUNK)], sidx)
        pltpu.sync_copy(dsth.at[pl.ds(base, CHUNK)], didx)
        pltpu.sync_copy(eah.at[pl.ds(base, CHUNK), :], eav)
        for hh in range(HEADS):
            pltpu.async_copy(tds[hh].at[didx], rd, sem).wait()
            pltpu.async_copy(tks[hh].at[sidx], rs, sem).wait()

            def edge_body(i, _):
                acc = eav[i, :] * rd[i, pl.ds(HID, 16)]
                for j in range(4):
                    acc = acc + rd[i, pl.ds(j * 16, 16)] * rs[i, pl.ds(j * 16, 16)]
                pb[i, :] = acc * 0.125
                return 0

            lax.fori_loop(0, CHUNK, edge_body, 0)
            pltpu.sync_copy(pb, pacc_out.at[hh, pl.ds(base, CHUNK), :])
        return 0

    lax.fori_loop(0, NCH, chunk_body, 0)


@functools.cache
def _k2():
    return functools.partial(
        pl.kernel,
        out_type=jax.ShapeDtypeStruct((HEADS, 2, 2, AROWS, ACCW), jnp.float32),
        mesh=_mesh(),
        scratch_types=[
            pltpu.VMEM((CHUNK,), jnp.int32),
            pltpu.VMEM((CHUNK,), jnp.int32),
            pltpu.VMEM((CHUNK,), jnp.int32),
            pltpu.VMEM((CHUNK, 16), jnp.float32),
            pltpu.VMEM((CHUNK, 2 * HID), jnp.float32),
            pltpu.VMEM((CHUNK,), jnp.float32),
            pltpu.VMEM((CHUNK, ACCW), jnp.float32),
            pltpu.VMEM((CHUNK, ACCW), jnp.float32),
            pltpu.VMEM_SHARED((AROWS, ACCW), jnp.float32),
            pltpu.SemaphoreType.DMA,
        ],
    )(_k2_body)


def _k2_body(tk0, tk1, tk2, tk3, eah, srch, dsth, exh,
             out, sidx, didx, didx2, eav, vrows, av, cb, zb, acc_sh, sem):
    cid = lax.axis_index("c")
    sid = lax.axis_index("s")
    wid = sid * 2 + cid
    base0 = wid * PERW
    tks = [tk0, tk1, tk2, tk3]

    def zrow(r, _):
        for j in range(ACCW // 16):
            zb[r, pl.ds(j * 16, 16)] = jnp.zeros((16,), jnp.float32)
            cb[r, pl.ds(j * 16, 16)] = jnp.zeros((16,), jnp.float32)
        return 0
    lax.fori_loop(0, CHUNK, zrow, 0)

    for hh in range(HEADS):
        for p in range(2):
            def zcopy(tk, _):
                pltpu.sync_copy(
                    zb,
                    acc_sh.at[pl.ds(sid * ROWS_PER_SUB + tk * CHUNK, CHUNK), :])
                return 0
            lax.fori_loop(0, ROWS_PER_SUB // CHUNK, zcopy, 0)
            plsc.subcore_barrier()

            def chunk_body(t, _):
                base = base0 + t * CHUNK
                pltpu.sync_copy(srch.at[pl.ds(base, CHUNK)], sidx)
                pltpu.sync_copy(dsth.at[pl.ds(base, CHUNK)], didx)
                pltpu.sync_copy(eah.at[pl.ds(base, CHUNK), :], eav)
                pltpu.async_copy(tks[hh].at[sidx], vrows, sem).wait()
                pltpu.sync_copy(exh.at[hh, pl.ds(base, CHUNK)], av)

                def remap(g, _):
                    d16 = didx[pl.ds(g * 16, 16)]
                    rel = d16 - p * HALF
                    ok = (rel >= 0) & (rel < HALF)
                    didx2[pl.ds(g * 16, 16)] = jnp.where(ok, rel, HALF)
                    return 0
                lax.fori_loop(0, CHUNK // 16, remap, 0)

                def group_body(g, _):
                    exv = av[pl.ds(g * 16, 16)]
                    for lz in range(16):
                        i = g * 16 + lz
                        ex = exv[lz]
                        for j in range(4):
                            cb[i, pl.ds(j * 16, 16)] = (
                                vrows[i, pl.ds(HID + j * 16, 16)] * ex)
                        cb[i, pl.ds(HID, 16)] = eav[i, :] * ex
                    return 0
                lax.fori_loop(0, CHUNK // 16, group_body, 0)
                pltpu.sync_copy(cb, acc_sh.at[didx2], add=True)
                return 0
            lax.fori_loop(0, NCH, chunk_body, 0)
            plsc.subcore_barrier()

            def drain(tk, _):
                rows = pl.ds(sid * ROWS_PER_SUB + tk * CHUNK, CHUNK)
                pltpu.sync_copy(acc_sh.at[rows, :], zb)
                pltpu.sync_copy(zb, out.at[hh, cid, p, rows, :])
                return 0
            lax.fori_loop(0, ROWS_PER_SUB // CHUNK, drain, 0)

            if hh < HEADS - 1 or p < 1:
                def rezero(r, _):
                    for j in range(ACCW // 16):
                        zb[r, pl.ds(j * 16, 16)] = jnp.zeros((16,), jnp.float32)
                    return 0
                lax.fori_loop(0, CHUNK, rezero, 0)


def kernel(x, edge_index, edge_attr, params):
    src_p = jnp.concatenate(
        [edge_index[0], jnp.full((EPAD - E,), DUMMY, jnp.int32)])
    dst_p = jnp.concatenate(
        [edge_index[1], jnp.full((EPAD - E,), DUMMY, jnp.int32)])
    ea16 = jnp.pad(edge_attr, ((0, EPAD - E), (0, 16 - EDGE_DIM)))
    ea16 = ea16.at[:, EDGE_DIM].set(1.0)
    ewp = jnp.concatenate(
        [params['edge_W'], params['edge_b'][None, :],
         jnp.zeros((16 - EDGE_DIM - 1, HID), jnp.float32)], axis=0)

    x_p = jnp.pad(x, ((0, NPAD - N), (0, HID - F_IN)))
    nw_p = jnp.pad(params['node_W'], ((0, HID - F_IN), (0, 0)))
    h = _dense(x_p, nw_p, params['node_b'], 512)

    for i in range(NUM_LAYERS):
        c = params['convs'][i]
        tds, tks, skip = _qkv(h, c, ewp)
        pacc = _k1()(*tds, *tks, ea16, src_p, dst_p)
        alpha, bmax = _alpha_max(pacc)
        ex = _exp(alpha, bmax).reshape(HEADS, EPAD)
        acc5 = _k2()(*tks, ea16, src_p, dst_p, ex)
        acc = jnp.concatenate(
            [acc5[:, :, 0, 0:HALF, :], acc5[:, :, 1, 0:HALF, :]], axis=2)
        h = _m2(acc, skip, c['We'], ewp, params['proj_W'], params['proj_b'],
                params['ln_g'][i], params['ln_b'][i], h,
                relu=(i < NUM_LAYERS - 1))

    return _logits(h[0:1], params['cls_W'], params['cls_b'])


# speedup vs baseline: 3.9529x; 1.2409x over previous
import functools

import jax
import jax.numpy as jnp
from jax import lax
from jax.experimental import pallas as pl
from jax.experimental.pallas import tpu as pltpu
from jax.experimental.pallas import tpu_sc as plsc

N = 10000
E = 160000
F_IN = 33
HID = 64
HEADS = 4
NUM_LAYERS = 3
NUM_CLASSES = 8
EDGE_DIM = 4
TEMP = 2.0

NPAD = 10240
EPAD = 163840
NW = 32
PERW = EPAD // NW
CHUNK = 128
NCH = PERW // CHUNK
ACCW = 128
HALF = 5120
AROWS = 6144
ROWS_PER_SUB = AROWS // 16
DUMMY = N


@functools.cache
def _mesh():
    return plsc.VectorSubcoreMesh(core_axis_name="c", subcore_axis_name="s")


def _dense_body(x_ref, w_ref, b_ref, o_ref):
    o_ref[...] = x_ref[...] @ w_ref[...] + b_ref[0:1, :]


def _dense(xp, w, b, rb):
    rows, cin = xp.shape
    cout = w.shape[1]
    return pl.pallas_call(
        _dense_body,
        grid=(rows // rb,),
        in_specs=[
            pl.BlockSpec((rb, cin), lambda i: (i, 0)),
            pl.BlockSpec((cin, cout), lambda i: (0, 0)),
            pl.BlockSpec((8, cout), lambda i: (0, 0)),
        ],
        out_specs=pl.BlockSpec((rb, cout), lambda i: (i, 0)),
        out_shape=jax.ShapeDtypeStruct((rows, cout), jnp.float32),
    )(xp, w, jnp.broadcast_to(b, (8, cout)))


def _qkv_body(h_ref, wq, wk, wv, ws, we, ewp, bq, bk, bv, bs, *outs):
    h = h_ref[...]
    q = h @ wq[...] + bq[0:1, :]
    k = h @ wk[...] + bk[0:1, :]
    v = h @ wv[...] + bv[0:1, :]
    outs[8][...] = h @ ws[...] + bs[0:1, :]
    we_a = we[...]
    ewp_a = ewp[...]
    rb = h.shape[0]
    for hh in range(HEADS):
        sl = slice(hh * HID, (hh + 1) * HID)
        qh = q[:, sl]
        qeh = lax.dot_general(qh, we_a[:, sl], (((1,), (1,)), ((), ())))
        qab = lax.dot_general(qeh, ewp_a, (((1,), (1,)), ((), ())))
        outs[hh][:, 0:HID] = qh
        outs[hh][:, HID:HID + 16] = qab
        outs[hh][:, HID + 16:2 * HID] = jnp.zeros((rb, 48), jnp.float32)
        outs[4 + hh][:, 0:HID] = k[:, sl]
        outs[4 + hh][:, HID:2 * HID] = v[:, sl]


def _qkv(h, c, ewp):
    rb = 512
    row = lambda i: (i, 0)
    full = lambda i: (0, 0)
    outs = pl.pallas_call(
        _qkv_body,
        grid=(NPAD // rb,),
        in_specs=[pl.BlockSpec((rb, HID), row)]
        + [pl.BlockSpec((HID, HEADS * HID), full)] * 5
        + [pl.BlockSpec((16, HID), full)]
        + [pl.BlockSpec((8, HEADS * HID), full)] * 4,
        out_specs=[pl.BlockSpec((rb, 2 * HID), row)] * 8
        + [pl.BlockSpec((rb, HEADS * HID), row)],
        out_shape=[jax.ShapeDtypeStruct((NPAD, 2 * HID), jnp.float32)] * 8
        + [jax.ShapeDtypeStruct((NPAD, HEADS * HID), jnp.float32)],
    )(
        h, c['Wq'], c['Wk'], c['Wv'], c['Ws'], c['We'], ewp,
        jnp.broadcast_to(c['bq'], (8, HEADS * HID)),
        jnp.broadcast_to(c['bk'], (8, HEADS * HID)),
        jnp.broadcast_to(c['bv'], (8, HEADS * HID)),
        jnp.broadcast_to(c['bs'], (8, HEADS * HID)),
    )
    return outs[0:4], outs[4:8], outs[8]


def _amax_body(p_ref, a_ref, m_ref):
    a = jnp.sum(p_ref[...][0], axis=-1)
    a_ref[...] = a[None, None, :]
    m_ref[...] = jnp.full((1, 1, 1, 128), jnp.max(a), jnp.float32)


def _alpha_max(pacc):
    eb = 2048
    nb = EPAD // eb
    return pl.pallas_call(
        _amax_body,
        grid=(HEADS, nb),
        in_specs=[pl.BlockSpec((1, eb, 16), lambda h, i: (h, i, 0))],
        out_specs=[
            pl.BlockSpec((1, 1, eb), lambda h, i: (h, 0, i)),
            pl.BlockSpec((1, 1, 1, 128), lambda h, i: (h, i, 0, 0)),
        ],
        out_shape=[
            jax.ShapeDtypeStruct((HEADS, 1, EPAD), jnp.float32),
            jax.ShapeDtypeStruct((HEADS, nb, 1, 128), jnp.float32),
        ],
    )(pacc)


def _exp_body(a_ref, m_ref, e_ref):
    gmax = jnp.max(m_ref[...])
    e_ref[...] = jnp.exp(a_ref[...] - gmax)


def _exp(alpha, bmax):
    eb = 2048
    nb = EPAD // eb
    return pl.pallas_call(
        _exp_body,
        grid=(HEADS, nb),
        in_specs=[
            pl.BlockSpec((1, 1, eb), lambda h, i: (h, 0, i)),
            pl.BlockSpec((1, nb, 1, 128), lambda h, i: (h, 0, 0, 0)),
        ],
        out_specs=pl.BlockSpec((1, 1, eb), lambda h, i: (h, 0, i)),
        out_shape=jax.ShapeDtypeStruct((HEADS, 1, EPAD), jnp.float32),
    )(alpha, bmax)


def _m2_body(acc_ref, skip_ref, we, ewp, pw, pb, g, bt, hin_ref, o_ref, *, relu):
    a = acc_ref[...]
    we_a = we[...]
    ewp_a = ewp[...]
    msgs = []
    for hh in range(HEADS):
        asum = a[hh, 0] + a[hh, 1]
        num = asum[:, 0:HID]
        sad = asum[:, HID:HID + 16]
        den = asum[:, HID + EDGE_DIM]
        unfold = ewp_a @ we_a[:, hh * HID:(hh + 1) * HID]
        m = (num + sad @ unfold) / (den[:, None] + 1e-16)
        msgs.append(m)
    out = jnp.concatenate(msgs, axis=1) + skip_ref[...]
    hn = out @ pw[...] + pb[0:1, :]
    mu = jnp.mean(hn, axis=1, keepdims=True)
    var = jnp.mean((hn - mu) ** 2, axis=1, keepdims=True)
    hn = (hn - mu) / jnp.sqrt(var + 1e-5) * g[0:1, :] + bt[0:1, :]
    hv = hin_ref[...] + hn
    if relu:
        hv = jnp.maximum(hv, 0.0)
    o_ref[...] = hv


def _m2(acc, skip, we, ewp, pw, pb, g, bt, hin, relu):
    rb = 512
    row = lambda i: (i, 0)
    full = lambda i: (0, 0)
    return pl.pallas_call(
        functools.partial(_m2_body, relu=relu),
        grid=(NPAD // rb,),
        in_specs=[
            pl.BlockSpec((HEADS, 2, rb, ACCW), lambda i: (0, 0, i, 0)),
            pl.BlockSpec((rb, HEADS * HID), row),
            pl.BlockSpec((HID, HEADS * HID), full),
            pl.BlockSpec((16, HID), full),
            pl.BlockSpec((HEADS * HID, HID), full),
            pl.BlockSpec((8, HID), full),
            pl.BlockSpec((8, HID), full),
            pl.BlockSpec((8, HID), full),
            pl.BlockSpec((rb, HID), row),
        ],
        out_specs=pl.BlockSpec((rb, HID), row),
        out_shape=jax.ShapeDtypeStruct((NPAD, HID), jnp.float32),
    )(acc, skip, we, ewp, pw,
      jnp.broadcast_to(pb, (8, HID)), jnp.broadcast_to(g, (8, HID)),
      jnp.broadcast_to(bt, (8, HID)), hin)


def _logits_body(h_ref, w_ref, b_ref, o_ref):
    o_ref[...] = (h_ref[...] @ w_ref[...] + b_ref[...]) / TEMP


def _logits(target, cw, cb):
    out = pl.pallas_call(
        _logits_body,
        out_shape=jax.ShapeDtypeStruct((8, NUM_CLASSES), jnp.float32),
    )(jnp.broadcast_to(target, (8, HID)), cw,
      jnp.broadcast_to(cb, (8, NUM_CLASSES)))
    return out[0:1]


@functools.cache
def _k1():
    return functools.partial(
        pl.kernel,
        out_type=jax.ShapeDtypeStruct((HEADS, EPAD, 16), jnp.float32),
        mesh=_mesh(),
        scratch_types=[
            pltpu.VMEM((CHUNK,), jnp.int32),
            pltpu.VMEM((CHUNK,), jnp.int32),
            pltpu.VMEM((CHUNK, 16), jnp.float32),
            pltpu.VMEM((CHUNK, 2 * HID), jnp.float32),
            pltpu.VMEM((CHUNK, 2 * HID), jnp.float32),
            pltpu.VMEM((CHUNK, 16), jnp.float32),
            pltpu.SemaphoreType.DMA,
        ],
    )(_k1_body)


def _k1_body(td0, td1, td2, td3, tk0, tk1, tk2, tk3, eah, srch, dsth,
             pacc_out, sidx, didx, eav, rd, rs, pb, sem):
    cid = lax.axis_index("c")
    sid = lax.axis_index("s")
    wid = sid * 2 + cid
    base0 = wid * PERW
    tds = [td0, td1, td2, td3]
    tks = [tk0, tk1, tk2, tk3]

    def chunk_body(t, _):
        base = base0 + t * CHUNK
        pltpu.sync_copy(srch.at[pl.ds(base, CHUNK)], sidx)
        pltpu.sync_copy(dsth.at[pl.ds(base, CHUNK)], didx)
        pltpu.sync_copy(eah.at[pl.ds(base, CHUNK), :], eav)
        for hh in range(HEADS):
            cp1 = pltpu.async_copy(tds[hh].at[didx], rd, sem)
            cp2 = pltpu.async_copy(tks[hh].at[sidx], rs, sem)
            cp1.wait()
            cp2.wait()

            def edge_body(i, _):
                acc = eav[i, :] * rd[i, pl.ds(HID, 16)]
                for j in range(4):
                    acc = acc + rd[i, pl.ds(j * 16, 16)] * rs[i, pl.ds(j * 16, 16)]
                pb[i, :] = acc * 0.125
                return 0

            lax.fori_loop(0, CHUNK, edge_body, 0)
            pltpu.sync_copy(pb, pacc_out.at[hh, pl.ds(base, CHUNK), :])
        return 0

    lax.fori_loop(0, NCH, chunk_body, 0)


@functools.cache
def _k2():
    return functools.partial(
        pl.kernel,
        out_type=jax.ShapeDtypeStruct((HEADS, 2, 2, AROWS, ACCW), jnp.float32),
        mesh=_mesh(),
        scratch_types=[
            pltpu.VMEM((CHUNK,), jnp.int32),
            pltpu.VMEM((CHUNK,), jnp.int32),
            pltpu.VMEM((CHUNK,), jnp.int32),
            pltpu.VMEM((CHUNK, 16), jnp.float32),
            pltpu.VMEM((CHUNK, 2 * HID), jnp.float32),
            pltpu.VMEM((CHUNK,), jnp.float32),
            pltpu.VMEM((CHUNK, ACCW), jnp.float32),
            pltpu.VMEM((CHUNK, ACCW), jnp.float32),
            pltpu.VMEM_SHARED((AROWS, ACCW), jnp.float32),
            pltpu.SemaphoreType.DMA,
        ],
    )(_k2_body)


def _k2_body(tk0, tk1, tk2, tk3, eah, srch, dsth, exh,
             out, sidx, didx, didx2, eav, vrows, av, cb, zb, acc_sh, sem):
    cid = lax.axis_index("c")
    sid = lax.axis_index("s")
    wid = sid * 2 + cid
    base0 = wid * PERW
    tks = [tk0, tk1, tk2, tk3]

    def zrow(r, _):
        for j in range(ACCW // 16):
            zb[r, pl.ds(j * 16, 16)] = jnp.zeros((16,), jnp.float32)
            cb[r, pl.ds(j * 16, 16)] = jnp.zeros((16,), jnp.float32)
        return 0
    lax.fori_loop(0, CHUNK, zrow, 0)

    for hh in range(HEADS):
        for p in range(2):
            def zcopy(tk, _):
                pltpu.sync_copy(
                    zb,
                    acc_sh.at[pl.ds(sid * ROWS_PER_SUB + tk * CHUNK, CHUNK), :])
                return 0
            lax.fori_loop(0, ROWS_PER_SUB // CHUNK, zcopy, 0)
            plsc.subcore_barrier()

            def chunk_body(t, _):
                base = base0 + t * CHUNK
                pltpu.sync_copy(srch.at[pl.ds(base, CHUNK)], sidx)
                cp = pltpu.async_copy(tks[hh].at[sidx], vrows, sem)
                pltpu.sync_copy(dsth.at[pl.ds(base, CHUNK)], didx)
                pltpu.sync_copy(eah.at[pl.ds(base, CHUNK), :], eav)
                pltpu.sync_copy(exh.at[hh, pl.ds(base, CHUNK)], av)
                cp.wait()

                def remap(g, _):
                    d16 = didx[pl.ds(g * 16, 16)]
                    rel = d16 - p * HALF
                    ok = (rel >= 0) & (rel < HALF)
                    didx2[pl.ds(g * 16, 16)] = jnp.where(ok, rel, HALF)
                    return 0
                lax.fori_loop(0, CHUNK // 16, remap, 0)

                def group_body(g, _):
                    exv = av[pl.ds(g * 16, 16)]
                    for lz in range(16):
                        i = g * 16 + lz
                        ex = exv[lz]
                        for j in range(4):
                            cb[i, pl.ds(j * 16, 16)] = (
                                vrows[i, pl.ds(HID + j * 16, 16)] * ex)
                        cb[i, pl.ds(HID, 16)] = eav[i, :] * ex
                    return 0
                lax.fori_loop(0, CHUNK // 16, group_body, 0)
                pltpu.sync_copy(cb, acc_sh.at[didx2], add=True)
                return 0
            lax.fori_loop(0, NCH, chunk_body, 0)
            plsc.subcore_barrier()

            def drain(tk, _):
                rows = pl.ds(sid * ROWS_PER_SUB + tk * CHUNK, CHUNK)
                pltpu.sync_copy(acc_sh.at[rows, :], zb)
                pltpu.sync_copy(zb, out.at[hh, cid, p, rows, :])
                return 0
            lax.fori_loop(0, ROWS_PER_SUB // CHUNK, drain, 0)

            if hh < HEADS - 1 or p < 1:
                def rezero(r, _):
                    for j in range(ACCW // 16):
                        zb[r, pl.ds(j * 16, 16)] = jnp.zeros((16,), jnp.float32)
                    return 0
                lax.fori_loop(0, CHUNK, rezero, 0)


def kernel(x, edge_index, edge_attr, params):
    src_p = jnp.concatenate(
        [edge_index[0], jnp.full((EPAD - E,), DUMMY, jnp.int32)])
    dst_p = jnp.concatenate(
        [edge_index[1], jnp.full((EPAD - E,), DUMMY, jnp.int32)])
    ea16 = jnp.pad(edge_attr, ((0, EPAD - E), (0, 16 - EDGE_DIM)))
    ea16 = ea16.at[:, EDGE_DIM].set(1.0)
    ewp = jnp.concatenate(
        [params['edge_W'], params['edge_b'][None, :],
         jnp.zeros((16 - EDGE_DIM - 1, HID), jnp.float32)], axis=0)

    x_p = jnp.pad(x, ((0, NPAD - N), (0, HID - F_IN)))
    nw_p = jnp.pad(params['node_W'], ((0, HID - F_IN), (0, 0)))
    h = _dense(x_p, nw_p, params['node_b'], 512)

    for i in range(NUM_LAYERS):
        c = params['convs'][i]
        tds, tks, skip = _qkv(h, c, ewp)
        pacc = _k1()(*tds, *tks, ea16, src_p, dst_p)
        alpha, bmax = _alpha_max(pacc)
        ex = _exp(alpha, bmax).reshape(HEADS, EPAD)
        acc5 = _k2()(*tks, ea16, src_p, dst_p, ex)
        acc = jnp.concatenate(
            [acc5[:, :, 0, 0:HALF, :], acc5[:, :, 1, 0:HALF, :]], axis=2)
        h = _m2(acc, skip, c['We'], ewp, params['proj_W'], params['proj_b'],
                params['ln_g'][i], params['ln_b'][i], h,
                relu=(i < NUM_LAYERS - 1))

    return _logits(h[0:1], params['cls_W'], params['cls_b'])


# speedup vs baseline: 4.0459x; 1.0235x over previous
import functools

import jax
import jax.numpy as jnp
from jax import lax
from jax.experimental import pallas as pl
from jax.experimental.pallas import tpu as pltpu
from jax.experimental.pallas import tpu_sc as plsc

N = 10000
E = 160000
F_IN = 33
HID = 64
HEADS = 4
NUM_LAYERS = 3
NUM_CLASSES = 8
EDGE_DIM = 4
TEMP = 2.0

NPAD = 10240
EPAD = 163840
NW = 32
PERW = EPAD // NW
CHUNK = 128
NCH = PERW // CHUNK
ACCW = 128
HALF = 5120
AROWS = 6144
ROWS_PER_SUB = AROWS // 16
DUMMY = N


@functools.cache
def _mesh():
    return plsc.VectorSubcoreMesh(core_axis_name="c", subcore_axis_name="s")


def _dense_body(x_ref, w_ref, b_ref, o_ref):
    o_ref[...] = x_ref[...] @ w_ref[...] + b_ref[0:1, :]


def _dense(xp, w, b, rb):
    rows, cin = xp.shape
    cout = w.shape[1]
    return pl.pallas_call(
        _dense_body,
        grid=(rows // rb,),
        in_specs=[
            pl.BlockSpec((rb, cin), lambda i: (i, 0)),
            pl.BlockSpec((cin, cout), lambda i: (0, 0)),
            pl.BlockSpec((8, cout), lambda i: (0, 0)),
        ],
        out_specs=pl.BlockSpec((rb, cout), lambda i: (i, 0)),
        out_shape=jax.ShapeDtypeStruct((rows, cout), jnp.float32),
    )(xp, w, jnp.broadcast_to(b, (8, cout)))


def _qkv_body(h_ref, wq, wk, wv, ws, we, ewp, bq, bk, bv, bs, *outs):
    h = h_ref[...]
    q = h @ wq[...] + bq[0:1, :]
    k = h @ wk[...] + bk[0:1, :]
    v = h @ wv[...] + bv[0:1, :]
    outs[8][...] = h @ ws[...] + bs[0:1, :]
    we_a = we[...]
    ewp_a = ewp[...]
    rb = h.shape[0]
    for hh in range(HEADS):
        sl = slice(hh * HID, (hh + 1) * HID)
        qh = q[:, sl]
        qeh = lax.dot_general(qh, we_a[:, sl], (((1,), (1,)), ((), ())))
        qab = lax.dot_general(qeh, ewp_a, (((1,), (1,)), ((), ())))
        outs[hh][:, 0:HID] = qh
        outs[hh][:, HID:HID + 16] = qab
        outs[hh][:, HID + 16:2 * HID] = jnp.zeros((rb, 48), jnp.float32)
        outs[4 + hh][:, 0:HID] = k[:, sl]
        outs[4 + hh][:, HID:2 * HID] = v[:, sl]


def _qkv(h, c, ewp):
    rb = 512
    row = lambda i: (i, 0)
    full = lambda i: (0, 0)
    outs = pl.pallas_call(
        _qkv_body,
        grid=(NPAD // rb,),
        in_specs=[pl.BlockSpec((rb, HID), row)]
        + [pl.BlockSpec((HID, HEADS * HID), full)] * 5
        + [pl.BlockSpec((16, HID), full)]
        + [pl.BlockSpec((8, HEADS * HID), full)] * 4,
        out_specs=[pl.BlockSpec((rb, 2 * HID), row)] * 8
        + [pl.BlockSpec((rb, HEADS * HID), row)],
        out_shape=[jax.ShapeDtypeStruct((NPAD, 2 * HID), jnp.float32)] * 8
        + [jax.ShapeDtypeStruct((NPAD, HEADS * HID), jnp.float32)],
    )(
        h, c['Wq'], c['Wk'], c['Wv'], c['Ws'], c['We'], ewp,
        jnp.broadcast_to(c['bq'], (8, HEADS * HID)),
        jnp.broadcast_to(c['bk'], (8, HEADS * HID)),
        jnp.broadcast_to(c['bv'], (8, HEADS * HID)),
        jnp.broadcast_to(c['bs'], (8, HEADS * HID)),
    )
    return outs[0:4], outs[4:8], outs[8]


def _amax_body(p_ref, a_ref, m_ref):
    a = jnp.sum(p_ref[...][0], axis=-1)
    a_ref[...] = a[None, None, :]
    m_ref[...] = jnp.full((1, 1, 1, 128), jnp.max(a), jnp.float32)


def _alpha_max(pacc):
    eb = 2048
    nb = EPAD // eb
    return pl.pallas_call(
        _amax_body,
        grid=(HEADS, nb),
        in_specs=[pl.BlockSpec((1, eb, 16), lambda h, i: (h, i, 0))],
        out_specs=[
            pl.BlockSpec((1, 1, eb), lambda h, i: (h, 0, i)),
            pl.BlockSpec((1, 1, 1, 128), lambda h, i: (h, i, 0, 0)),
        ],
        out_shape=[
            jax.ShapeDtypeStruct((HEADS, 1, EPAD), jnp.float32),
            jax.ShapeDtypeStruct((HEADS, nb, 1, 128), jnp.float32),
        ],
    )(pacc)


def _exp_body(a_ref, m_ref, e_ref):
    gmax = jnp.max(m_ref[...])
    e_ref[...] = jnp.exp(a_ref[...] - gmax)


def _exp(alpha, bmax):
    eb = 2048
    nb = EPAD // eb
    return pl.pallas_call(
        _exp_body,
        grid=(HEADS, nb),
        in_specs=[
            pl.BlockSpec((1, 1, eb), lambda h, i: (h, 0, i)),
            pl.BlockSpec((1, nb, 1, 128), lambda h, i: (h, 0, 0, 0)),
        ],
        out_specs=pl.BlockSpec((1, 1, eb), lambda h, i: (h, 0, i)),
        out_shape=jax.ShapeDtypeStruct((HEADS, 1, EPAD), jnp.float32),
    )(alpha, bmax)


def _m2_body(acc_ref, skip_ref, we, ewp, pw, pb, g, bt, hin_ref, o_ref, *, relu):
    a = acc_ref[...]
    we_a = we[...]
    ewp_a = ewp[...]
    msgs = []
    for hh in range(HEADS):
        asum = a[hh, 0] + a[hh, 1]
        num = asum[:, 0:HID]
        sad = asum[:, HID:HID + 16]
        den = asum[:, HID + EDGE_DIM]
        unfold = ewp_a @ we_a[:, hh * HID:(hh + 1) * HID]
        m = (num + sad @ unfold) / (den[:, None] + 1e-16)
        msgs.append(m)
    out = jnp.concatenate(msgs, axis=1) + skip_ref[...]
    hn = out @ pw[...] + pb[0:1, :]
    mu = jnp.mean(hn, axis=1, keepdims=True)
    var = jnp.mean((hn - mu) ** 2, axis=1, keepdims=True)
    hn = (hn - mu) / jnp.sqrt(var + 1e-5) * g[0:1, :] + bt[0:1, :]
    hv = hin_ref[...] + hn
    if relu:
        hv = jnp.maximum(hv, 0.0)
    o_ref[...] = hv


def _m2(acc, skip, we, ewp, pw, pb, g, bt, hin, relu):
    rb = 512
    row = lambda i: (i, 0)
    full = lambda i: (0, 0)
    return pl.pallas_call(
        functools.partial(_m2_body, relu=relu),
        grid=(NPAD // rb,),
        in_specs=[
            pl.BlockSpec((HEADS, 2, rb, ACCW), lambda i: (0, 0, i, 0)),
            pl.BlockSpec((rb, HEADS * HID), row),
            pl.BlockSpec((HID, HEADS * HID), full),
            pl.BlockSpec((16, HID), full),
            pl.BlockSpec((HEADS * HID, HID), full),
            pl.BlockSpec((8, HID), full),
            pl.BlockSpec((8, HID), full),
            pl.BlockSpec((8, HID), full),
            pl.BlockSpec((rb, HID), row),
        ],
        out_specs=pl.BlockSpec((rb, HID), row),
        out_shape=jax.ShapeDtypeStruct((NPAD, HID), jnp.float32),
    )(acc, skip, we, ewp, pw,
      jnp.broadcast_to(pb, (8, HID)), jnp.broadcast_to(g, (8, HID)),
      jnp.broadcast_to(bt, (8, HID)), hin)


def _logits_body(h_ref, w_ref, b_ref, o_ref):
    o_ref[...] = (h_ref[...] @ w_ref[...] + b_ref[...]) / TEMP


def _logits(target, cw, cb):
    out = pl.pallas_call(
        _logits_body,
        out_shape=jax.ShapeDtypeStruct((8, NUM_CLASSES), jnp.float32),
    )(jnp.broadcast_to(target, (8, HID)), cw,
      jnp.broadcast_to(cb, (8, NUM_CLASSES)))
    return out[0:1]


@functools.cache
def _k1():
    return functools.partial(
        pl.kernel,
        out_type=jax.ShapeDtypeStruct((HEADS, EPAD, 16), jnp.float32),
        mesh=_mesh(),
        scratch_types=[
            pltpu.VMEM((CHUNK,), jnp.int32),
            pltpu.VMEM((CHUNK,), jnp.int32),
            pltpu.VMEM((CHUNK, 16), jnp.float32),
            pltpu.VMEM((CHUNK, 2 * HID), jnp.float32),
            pltpu.VMEM((CHUNK, 2 * HID), jnp.float32),
            pltpu.VMEM((CHUNK, 2 * HID), jnp.float32),
            pltpu.VMEM((CHUNK, 2 * HID), jnp.float32),
            pltpu.VMEM((CHUNK, 16), jnp.float32),
            pltpu.SemaphoreType.DMA,
        ],
    )(_k1_body)


def _k1_body(td0, td1, td2, td3, tk0, tk1, tk2, tk3, eah, srch, dsth,
             pacc_out, sidx, didx, eav, rda, rdb, rsa, rsb, pb, sem):
    cid = lax.axis_index("c")
    sid = lax.axis_index("s")
    wid = sid * 2 + cid
    base0 = wid * PERW
    tds = [td0, td1, td2, td3]
    tks = [tk0, tk1, tk2, tk3]
    rds = [rda, rdb]
    rss = [rsa, rsb]

    def chunk_body(t, _):
        base = base0 + t * CHUNK
        pltpu.sync_copy(srch.at[pl.ds(base, CHUNK)], sidx)
        pltpu.sync_copy(dsth.at[pl.ds(base, CHUNK)], didx)
        pltpu.sync_copy(eah.at[pl.ds(base, CHUNK), :], eav)
        cps = [pltpu.async_copy(tds[0].at[didx], rds[0], sem),
               pltpu.async_copy(tks[0].at[sidx], rss[0], sem)]
        for hh in range(HEADS):
            buf = hh % 2
            rd = rds[buf]
            rs = rss[buf]
            cps[0].wait()
            cps[1].wait()
            if hh < HEADS - 1:
                nxt = 1 - buf
                cps = [pltpu.async_copy(tds[hh + 1].at[didx], rds[nxt], sem),
                       pltpu.async_copy(tks[hh + 1].at[sidx], rss[nxt], sem)]

            def edge_body(i, _):
                acc = eav[i, :] * rd[i, pl.ds(HID, 16)]
                for j in range(4):
                    acc = acc + rd[i, pl.ds(j * 16, 16)] * rs[i, pl.ds(j * 16, 16)]
                pb[i, :] = acc * 0.125
                return 0

            lax.fori_loop(0, CHUNK, edge_body, 0)
            pltpu.sync_copy(pb, pacc_out.at[hh, pl.ds(base, CHUNK), :])
        return 0

    lax.fori_loop(0, NCH, chunk_body, 0)


@functools.cache
def _k2():
    return functools.partial(
        pl.kernel,
        out_type=jax.ShapeDtypeStruct((HEADS, 2, 2, AROWS, ACCW), jnp.float32),
        mesh=_mesh(),
        scratch_types=[
            pltpu.VMEM((CHUNK,), jnp.int32),
            pltpu.VMEM((CHUNK,), jnp.int32),
            pltpu.VMEM((CHUNK,), jnp.int32),
            pltpu.VMEM((CHUNK, 16), jnp.float32),
            pltpu.VMEM((CHUNK, 2 * HID), jnp.float32),
            pltpu.VMEM((CHUNK,), jnp.float32),
            pltpu.VMEM((CHUNK, ACCW), jnp.float32),
            pltpu.VMEM((CHUNK, ACCW), jnp.float32),
            pltpu.VMEM_SHARED((AROWS, ACCW), jnp.float32),
            pltpu.SemaphoreType.DMA,
        ],
    )(_k2_body)


def _k2_body(tk0, tk1, tk2, tk3, eah, srch, dsth, exh,
             out, sidx, didx, didx2, eav, vrows, av, cb, zb, acc_sh, sem):
    cid = lax.axis_index("c")
    sid = lax.axis_index("s")
    wid = sid * 2 + cid
    base0 = wid * PERW
    tks = [tk0, tk1, tk2, tk3]

    def zrow(r, _):
        for j in range(ACCW // 16):
            zb[r, pl.ds(j * 16, 16)] = jnp.zeros((16,), jnp.float32)
            cb[r, pl.ds(j * 16, 16)] = jnp.zeros((16,), jnp.float32)
        return 0
    lax.fori_loop(0, CHUNK, zrow, 0)

    for hh in range(HEADS):
        for p in range(2):
            def zcopy(tk, _):
                pltpu.sync_copy(
                    zb,
                    acc_sh.at[pl.ds(sid * ROWS_PER_SUB + tk * CHUNK, CHUNK), :])
                return 0
            lax.fori_loop(0, ROWS_PER_SUB // CHUNK, zcopy, 0)
            plsc.subcore_barrier()

            def chunk_body(t, _):
                base = base0 + t * CHUNK
                pltpu.sync_copy(srch.at[pl.ds(base, CHUNK)], sidx)
                cp = pltpu.async_copy(tks[hh].at[sidx], vrows, sem)
                pltpu.sync_copy(dsth.at[pl.ds(base, CHUNK)], didx)
                pltpu.sync_copy(eah.at[pl.ds(base, CHUNK), :], eav)
                pltpu.sync_copy(exh.at[hh, pl.ds(base, CHUNK)], av)
                cp.wait()

                def remap(g, _):
                    d16 = didx[pl.ds(g * 16, 16)]
                    rel = d16 - p * HALF
                    ok = (rel >= 0) & (rel < HALF)
                    didx2[pl.ds(g * 16, 16)] = jnp.where(ok, rel, HALF)
                    return 0
                lax.fori_loop(0, CHUNK // 16, remap, 0)

                def group_body(g, _):
                    exv = av[pl.ds(g * 16, 16)]
                    for lz in range(16):
                        i = g * 16 + lz
                        ex = exv[lz]
                        for j in range(4):
                            cb[i, pl.ds(j * 16, 16)] = (
                                vrows[i, pl.ds(HID + j * 16, 16)] * ex)
                        cb[i, pl.ds(HID, 16)] = eav[i, :] * ex
                    return 0
                lax.fori_loop(0, CHUNK // 16, group_body, 0)
                pltpu.sync_copy(cb, acc_sh.at[didx2], add=True)
                return 0
            lax.fori_loop(0, NCH, chunk_body, 0)
            plsc.subcore_barrier()

            def drain(tk, _):
                rows = pl.ds(sid * ROWS_PER_SUB + tk * CHUNK, CHUNK)
                pltpu.sync_copy(acc_sh.at[rows, :], zb)
                pltpu.sync_copy(zb, out.at[hh, cid, p, rows, :])
                return 0
            lax.fori_loop(0, ROWS_PER_SUB // CHUNK, drain, 0)

            if hh < HEADS - 1 or p < 1:
                def rezero(r, _):
                    for j in range(ACCW // 16):
                        zb[r, pl.ds(j * 16, 16)] = jnp.zeros((16,), jnp.float32)
                    return 0
                lax.fori_loop(0, CHUNK, rezero, 0)


def kernel(x, edge_index, edge_attr, params):
    src_p = jnp.concatenate(
        [edge_index[0], jnp.full((EPAD - E,), DUMMY, jnp.int32)])
    dst_p = jnp.concatenate(
        [edge_index[1], jnp.full((EPAD - E,), DUMMY, jnp.int32)])
    ea16 = jnp.pad(edge_attr, ((0, EPAD - E), (0, 16 - EDGE_DIM)))
    ea16 = ea16.at[:, EDGE_DIM].set(1.0)
    ewp = jnp.concatenate(
        [params['edge_W'], params['edge_b'][None, :],
         jnp.zeros((16 - EDGE_DIM - 1, HID), jnp.float32)], axis=0)

    x_p = jnp.pad(x, ((0, NPAD - N), (0, HID - F_IN)))
    nw_p = jnp.pad(params['node_W'], ((0, HID - F_IN), (0, 0)))
    h = _dense(x_p, nw_p, params['node_b'], 512)

    for i in range(NUM_LAYERS):
        c = params['convs'][i]
        tds, tks, skip = _qkv(h, c, ewp)
        pacc = _k1()(*tds, *tks, ea16, src_p, dst_p)
        alpha, bmax = _alpha_max(pacc)
        ex = _exp(alpha, bmax).reshape(HEADS, EPAD)
        acc5 = _k2()(*tks, ea16, src_p, dst_p, ex)
        acc = jnp.concatenate(
            [acc5[:, :, 0, 0:HALF, :], acc5[:, :, 1, 0:HALF, :]], axis=2)
        h = _m2(acc, skip, c['We'], ewp, params['proj_W'], params['proj_b'],
                params['ln_g'][i], params['ln_b'][i], h,
                relu=(i < NUM_LAYERS - 1))

    return _logits(h[0:1], params['cls_W'], params['cls_b'])
